# Initial kernel scaffold; baseline (speedup 1.0000x reference)
import jax
import jax.numpy as jnp
from jax.experimental import pallas as pl


def kernel(features, tables):
    raise NotImplementedError("write your pallas kernel here")


# speedup vs baseline: 1.2067x; 1.2067x over previous
import jax
import jax.numpy as jnp
from jax import lax
from jax.experimental import pallas as pl
from jax.experimental.pallas import tpu as pltpu
from jax.experimental.pallas import tpu_sc as plsc

NUM_FIELDS = 26
VOCAB = 100000
EMBED_DIM = 32
BATCH = 16384

NC, NS = 2, 16
NW = NC * NS
ROWS = BATCH * NUM_FIELDS
RPW = ROWS // NW
CHUNK = 1664
NCHUNK = RPW // CHUNK
GSLICE = 128
PAT = 13


def _body(feat_hbm, table_hbm, out_hbm, idx_v, rows_v, pat_v, sem):
    wid = lax.axis_index("s") * NC + lax.axis_index("c")
    for k in range(PAT):
        p = lax.iota(jnp.int32, 16) + (16 * k)
        pat_v[k] = (p % NUM_FIELDS) * VOCAB

    def chunk(c, carry):
        base = wid * RPW + c * CHUNK
        pltpu.sync_copy(feat_hbm.at[pl.ds(base, CHUNK)], idx_v)
        for k in range(CHUNK // 16):
            sl = pl.ds(k * 16, 16)
            idx_v[sl] = idx_v[sl] + pat_v[k % PAT]
        copies = [
            pltpu.async_copy(
                table_hbm.at[idx_v.at[pl.ds(r * GSLICE, GSLICE)]],
                rows_v.at[pl.ds(r * GSLICE, GSLICE)],
                sem,
            )
            for r in range(CHUNK // GSLICE)
        ]
        for cp in copies:
            cp.wait()
        pltpu.sync_copy(rows_v, out_hbm.at[pl.ds(base, CHUNK)])
        return carry

    lax.fori_loop(0, NCHUNK, chunk, 0)


_gather = pl.kernel(
    _body,
    out_type=jax.ShapeDtypeStruct((ROWS, EMBED_DIM), jnp.float32),
    mesh=plsc.VectorSubcoreMesh(core_axis_name="c", subcore_axis_name="s"),
    compiler_params=pltpu.CompilerParams(use_tc_tiling_on_sc=False),
    scratch_types=[
        pltpu.VMEM((CHUNK,), jnp.int32),
        pltpu.VMEM((CHUNK, EMBED_DIM), jnp.float32),
        pltpu.VMEM((PAT, 16), jnp.int32),
        pltpu.SemaphoreType.DMA,
    ],
)


def kernel(features, tables):
    feats = features.reshape(ROWS).astype(jnp.int32)
    table = tables.reshape(NUM_FIELDS * VOCAB, EMBED_DIM)
    out = _gather(feats, table)
    return out.reshape(BATCH, NUM_FIELDS * EMBED_DIM)


# speedup vs baseline: 1.3565x; 1.1242x over previous
import jax
import jax.numpy as jnp
from jax import lax
from jax.experimental import pallas as pl
from jax.experimental.pallas import tpu as pltpu
from jax.experimental.pallas import tpu_sc as plsc

NUM_FIELDS = 26
VOCAB = 100000
EMBED_DIM = 32
BATCH = 16384

NC, NS = 2, 16
NW = NC * NS
CBLK = BATCH // 128
NCHUNK = NUM_FIELDS * CBLK
CH_PER_W = NCHUNK // NW

_MESH = plsc.VectorSubcoreMesh(core_axis_name="c", subcore_axis_name="s")


def _gather_body(fT_hbm, tT_hbm, o_hbm, idx_v, o_v, sem):
    wid = lax.axis_index("s") * NC + lax.axis_index("c")

    def chunk(c, carry):
        cid = wid * CH_PER_W + c
        f = cid >> 7
        cb = cid & 127
        pltpu.sync_copy(fT_hbm.at[f, pl.ds(cb * 128, 128)], idx_v)
        copies = [
            pltpu.async_copy(tT_hbm.at[f, e].at[idx_v], o_v.at[e], sem)
            for e in range(EMBED_DIM)
        ]
        for cp in copies:
            cp.wait()
        pltpu.sync_copy(o_v, o_hbm.at[pl.ds(32 * f, 32), pl.ds(cb * 128, 128)])
        return carry

    lax.fori_loop(0, CH_PER_W, chunk, 0)


_gather_call = pl.kernel(
    _gather_body,
    out_type=jax.ShapeDtypeStruct((NUM_FIELDS * EMBED_DIM, BATCH),
                                  jnp.float32),
    mesh=_MESH,
    compiler_params=pltpu.CompilerParams(use_tc_tiling_on_sc=False),
    scratch_types=[
        pltpu.VMEM((128,), jnp.int32),
        pltpu.VMEM((EMBED_DIM, 128), jnp.float32),
        pltpu.SemaphoreType.DMA,
    ],
)


def kernel(features, tables):
    tT = tables.transpose(0, 2, 1)
    fT = features.astype(jnp.int32).T
    o = _gather_call(fT, tT)
    return o.T


# speedup vs baseline: 1.5090x; 1.1124x over previous
import jax
import jax.numpy as jnp
from jax import lax
from jax.experimental import pallas as pl
from jax.experimental.pallas import tpu as pltpu
from jax.experimental.pallas import tpu_sc as plsc

NUM_FIELDS = 26
VOCAB = 100000
EMBED_DIM = 32
BATCH = 16384

NC, NS = 2, 16
NW = NC * NS
BBLK = 1024
CBLK = BATCH // BBLK
NCHUNK = NUM_FIELDS * CBLK
CH_PER_W = NCHUNK // NW

_MESH = plsc.VectorSubcoreMesh(core_axis_name="c", subcore_axis_name="s")


def _gather_body(fT_hbm, tT_hbm, o_hbm, idx_v, o_v, sem):
    wid = lax.axis_index("s") * NC + lax.axis_index("c")

    def chunk(c, carry):
        cid = wid * CH_PER_W + c
        f = cid // CBLK
        cb = cid % CBLK
        pltpu.sync_copy(fT_hbm.at[f, pl.ds(cb * BBLK, BBLK)], idx_v)
        copies = [
            pltpu.async_copy(tT_hbm.at[f, e].at[idx_v], o_v.at[e], sem)
            for e in range(EMBED_DIM)
        ]
        for cp in copies:
            cp.wait()
        pltpu.sync_copy(o_v, o_hbm.at[pl.ds(32 * f, 32), pl.ds(cb * BBLK, BBLK)])
        return carry

    lax.fori_loop(0, CH_PER_W, chunk, 0)


_gather_call = pl.kernel(
    _gather_body,
    out_type=jax.ShapeDtypeStruct((NUM_FIELDS * EMBED_DIM, BATCH),
                                  jnp.float32),
    mesh=_MESH,
    compiler_params=pltpu.CompilerParams(use_tc_tiling_on_sc=False),
    scratch_types=[
        pltpu.VMEM((BBLK,), jnp.int32),
        pltpu.VMEM((EMBED_DIM, BBLK), jnp.float32),
        pltpu.SemaphoreType.DMA,
    ],
)


def kernel(features, tables):
    tT = tables.transpose(0, 2, 1)
    fT = features.astype(jnp.int32).T
    o = _gather_call(fT, tT)
    return o.T


# speedup vs baseline: 1.6190x; 1.0729x over previous
import jax
import jax.numpy as jnp
from jax import lax
from jax.experimental import pallas as pl
from jax.experimental.pallas import tpu as pltpu
from jax.experimental.pallas import tpu_sc as plsc

NUM_FIELDS = 26
VOCAB = 100000
EMBED_DIM = 32
BATCH = 16384

NC, NS = 2, 16
NW = NC * NS
BBLK = 1024
CBLK = BATCH // BBLK
GROUPS = (2, 4, 4, 4, 4, 4, 4)

_MESH = plsc.VectorSubcoreMesh(core_axis_name="c", subcore_axis_name="s")


def _make_gather(g):
    ch_per_w = g * CBLK // NW

    def body(fT_hbm, tT_hbm, o_hbm, idx_v, o_v, sem):
        wid = lax.axis_index("s") * NC + lax.axis_index("c")

        def chunk(c, carry):
            cid = wid * ch_per_w + c
            f = cid // CBLK
            cb = cid % CBLK
            pltpu.sync_copy(fT_hbm.at[f, pl.ds(cb * BBLK, BBLK)], idx_v)
            copies = [
                pltpu.async_copy(tT_hbm.at[f, e].at[idx_v], o_v.at[e], sem)
                for e in range(EMBED_DIM)
            ]
            for cp in copies:
                cp.wait()
            pltpu.sync_copy(
                o_v, o_hbm.at[pl.ds(32 * f, 32), pl.ds(cb * BBLK, BBLK)])
            return carry

        lax.fori_loop(0, ch_per_w, chunk, 0)

    return pl.kernel(
        body,
        out_type=jax.ShapeDtypeStruct((g * EMBED_DIM, BATCH), jnp.float32),
        mesh=_MESH,
        compiler_params=pltpu.CompilerParams(use_tc_tiling_on_sc=False),
        scratch_types=[
            pltpu.VMEM((BBLK,), jnp.int32),
            pltpu.VMEM((EMBED_DIM, BBLK), jnp.float32),
            pltpu.SemaphoreType.DMA,
        ],
    )


_CALLS = {g: _make_gather(g) for g in set(GROUPS)}


def kernel(features, tables):
    tT = tables.transpose(0, 2, 1)
    fT = features.astype(jnp.int32).T
    outs = []
    off = 0
    for g in GROUPS:
        outs.append(_CALLS[g](fT[off:off + g], tT[off:off + g]))
        off += g
    o = jnp.concatenate(outs, axis=0)
    return o.T
